# baseline (speedup 1.0000x reference)
import functools

import jax
import jax.numpy as jnp
from jax import lax
from jax.experimental import pallas as pl
from jax.experimental.pallas import tpu as pltpu
from jax.experimental.pallas import tpu_sc as plsc

CHUNK = 64
DCHUNK = 128


def _make_deg_kernel(N1, NC, NS, CPW):
  NW = NC * NS
  stripe = N1 // NS
  mesh = plsc.VectorSubcoreMesh(core_axis_name="c", subcore_axis_name="s")

  @functools.partial(
      pl.kernel,
      out_type=jax.ShapeDtypeStruct((NC, N1), jnp.float32),
      mesh=mesh,
      scratch_types=[
          pltpu.VMEM((CPW, DCHUNK), jnp.int32),
          pltpu.VMEM((DCHUNK,), jnp.float32),
          pltpu.VMEM_SHARED((N1,), jnp.float32),
      ],
  )
  def deg_kernel(dst_hbm, zvec_hbm, deg_out, idx_v, ones_v, deg_sh):
    cid = lax.axis_index("c")
    sid = lax.axis_index("s")
    wid = sid * NC + cid
    for i in range(DCHUNK // 16):
      ones_v[pl.ds(i * 16, 16)] = jnp.ones((16,), jnp.float32)
    pltpu.sync_copy(zvec_hbm.at[pl.ds(sid * stripe, stripe)],
                    deg_sh.at[pl.ds(sid * stripe, stripe)])
    pltpu.sync_copy(dst_hbm.at[wid], idx_v)
    plsc.subcore_barrier()

    def body(j, carry):
      pltpu.sync_copy(ones_v, deg_sh.at[idx_v.at[j]], add=True)
      return carry

    lax.fori_loop(0, CPW, body, 0)
    plsc.subcore_barrier()
    pltpu.sync_copy(deg_sh.at[pl.ds(sid * stripe, stripe)],
                    deg_out.at[cid, pl.ds(sid * stripe, stripe)])

  return deg_kernel


def _make_prop_kernel(N1, N1a, H, NC, NS, CPW):
  NW = NC * NS
  stripe = N1 // NS
  stripe_a = N1a // NS
  assert CPW % 16 == 0
  HCPW = CPW // 4
  mesh = plsc.VectorSubcoreMesh(core_axis_name="c", subcore_axis_name="s")

  @functools.partial(
      pl.kernel,
      out_type=(
          jax.ShapeDtypeStruct((NC, N1a, H), jnp.float32),
          jax.ShapeDtypeStruct((NC, N1), jnp.float32),
      ),
      mesh=mesh,
      scratch_types=[
          pltpu.VMEM((HCPW, CHUNK), jnp.int32),
          pltpu.VMEM((HCPW, CHUNK), jnp.int32),
          pltpu.VMEM((CHUNK, H), jnp.float32),
          pltpu.VMEM((CHUNK, H), jnp.float32),
          pltpu.VMEM((CHUNK, H), jnp.float32),
          pltpu.VMEM((CHUNK, H), jnp.float32),
          pltpu.VMEM((HCPW, CHUNK), jnp.float32),
          pltpu.VMEM_SHARED((N1a, H), jnp.float32),
          pltpu.VMEM_SHARED((N1,), jnp.float32),
          [pltpu.SemaphoreType.DMA] * 4,
          [pltpu.SemaphoreType.DMA] * 4,
          [pltpu.SemaphoreType.DMA] * 2,
          pltpu.SemaphoreType.DMA,
      ],
  )
  def prop_kernel(src_hbm, dst_hbm, g_hbm, dinv_hbm, zrows_hbm, zvec_hbm,
                  acc_out, t_out, src_v, dst_v, rows0, rows1, rows2, rows3,
                  dvals_v, acc_sh, t_sh, gsems, ssems, dsems, tsem):
    cid = lax.axis_index("c")
    sid = lax.axis_index("s")
    wid = sid * NC + cid
    pltpu.sync_copy(zrows_hbm.at[pl.ds(sid * stripe_a, stripe_a)],
                    acc_sh.at[pl.ds(sid * stripe_a, stripe_a)])
    pltpu.sync_copy(zvec_hbm.at[pl.ds(sid * stripe, stripe)],
                    t_sh.at[pl.ds(sid * stripe, stripe)])
    plsc.subcore_barrier()

    rows = (rows0, rows1, rows2, rows3)

    for h in range(4):
      pltpu.sync_copy(src_hbm.at[wid, pl.ds(h * HCPW, HCPW)], src_v)
      pltpu.sync_copy(dst_hbm.at[wid, pl.ds(h * HCPW, HCPW)], dst_v)
      for p in range(2):
        pltpu.async_copy(dinv_hbm.at[dst_v.at[p]], dvals_v.at[p], dsems[p])
        pltpu.async_copy(g_hbm.at[src_v.at[p]], rows[p], gsems[p])

      def body(i, carry):
        for p in range(4):
          j = 4 * i + p
          q = (p + 2) % 4
          pltpu.make_async_copy(dinv_hbm.at[dst_v.at[j]], dvals_v.at[j],
                                dsems[p % 2]).wait()
          pltpu.async_copy(dvals_v.at[j], t_sh.at[src_v.at[j]], tsem,
                           add=True)

          @pl.when(j + 2 < HCPW)
          def _():
            pltpu.async_copy(dinv_hbm.at[dst_v.at[j + 2]], dvals_v.at[j + 2],
                             dsems[p % 2])

          pltpu.make_async_copy(g_hbm.at[src_v.at[j]], rows[p],
                                gsems[p]).wait()
          pltpu.async_copy(rows[p], acc_sh.at[dst_v.at[j]], ssems[p],
                           add=True)

          @pl.when(j + 2 < HCPW)
          def _():
            @pl.when(j >= 2)
            def _():
              pltpu.make_async_copy(rows[q], acc_sh.at[dst_v.at[j]],
                                    ssems[q]).wait()

            pltpu.async_copy(g_hbm.at[src_v.at[j + 2]], rows[q], gsems[q])

        return carry

      lax.fori_loop(0, HCPW // 4, body, 0)

      for p in range(4):
        pltpu.make_async_copy(rows[p], acc_sh.at[dst_v.at[0]],
                              ssems[p]).wait()

      def drain(jj, carry):
        pltpu.make_async_copy(zvec_hbm.at[pl.ds(0, CHUNK)], dvals_v.at[0],
                              tsem).wait()
        return carry

      lax.fori_loop(0, HCPW, drain, 0)

    plsc.subcore_barrier()
    pltpu.sync_copy(acc_sh.at[pl.ds(sid * stripe_a, stripe_a)],
                    acc_out.at[cid, pl.ds(sid * stripe_a, stripe_a)])
    pltpu.sync_copy(t_sh.at[pl.ds(sid * stripe, stripe)],
                    t_out.at[cid, pl.ds(sid * stripe, stripe)])

  return prop_kernel


def _mm_scale_body(x_ref, w_ref, degp_ref, g_ref, dinv_ref):
  deg = degp_ref[0] + degp_ref[1] + 1.0
  dinv = lax.rsqrt(deg)
  h = jnp.dot(x_ref[...], w_ref[...], preferred_element_type=jnp.float32,
              precision=lax.Precision.HIGHEST)
  g_ref[...] = h * dinv
  dinv_ref[...] = dinv


def _combine_body(N, NB, accp_ref, g_ref, dinv_ref, tp_ref, b1_ref, w2_ref,
                  b2_ref, v_ref, out_ref):
  i = pl.program_id(0)
  R = g_ref.shape[0]
  dinv = dinv_ref[...]
  accsum = accp_ref[0] + accp_ref[1]
  out1 = dinv * (accsum + g_ref[...]) + b1_ref[...]
  h1r = jnp.maximum(out1, 0.0)
  t = tp_ref[0] + tp_ref[1]
  c = dinv * t + dinv * dinv
  rowid = i * R + lax.broadcasted_iota(jnp.int32, (R, 1), 0)
  c = jnp.where(rowid < N, c, 0.0)
  contrib = jnp.sum(c * h1r, axis=0, keepdims=True)

  @pl.when(i == 0)
  def _():
    v_ref[...] = jnp.zeros_like(v_ref)

  v_ref[...] += contrib

  @pl.when(i == NB - 1)
  def _():
    out_ref[...] = jnp.dot(v_ref[...] * (1.0 / N), w2_ref[...],
                           preferred_element_type=jnp.float32,
                           precision=lax.Precision.HIGHEST) + b2_ref[...]


def kernel(x, edge_index, W1, b1, W2, b2):
  N, D = x.shape
  H = W1.shape[1]
  E = edge_index.shape[1]

  info = plsc.get_sparse_core_info()
  NC, NS = info.num_cores, info.num_subcores
  NW = NC * NS
  CPW = -(-E // (NW * CHUNK))
  CPW = -(-CPW // 16) * 16
  E_pad = NW * CHUNK * CPW
  CPWD = E_pad // (NW * DCHUNK)
  assert CPWD * NW * DCHUNK == E_pad

  R = 1280
  N1 = -(-(N + 1) // R) * R
  assert N1 % (NS * CHUNK) == 0
  Ra = 1264
  N1a = -(-(N + 1) // Ra) * Ra
  assert N1a % (NS * 8) == 0 and N1a <= N1

  src = edge_index[0].astype(jnp.int32)
  dst = edge_index[1].astype(jnp.int32)
  pad = jnp.full((E_pad - E,), N, jnp.int32)
  src3 = jnp.concatenate([src, pad]).reshape(NW, CPW, CHUNK)
  dst3 = jnp.concatenate([dst, pad]).reshape(NW, CPW, CHUNK)
  zvec = jnp.zeros((N1,), jnp.float32)
  zrows = jnp.zeros((N1a, H), jnp.float32)
  xp = jnp.concatenate([x, jnp.zeros((N1 - N, D), x.dtype)], axis=0)

  dst3d = jnp.concatenate([dst, pad]).reshape(NW, CPWD, DCHUNK)
  deg_part = _make_deg_kernel(N1, NC, NS, CPWD)(dst3d, zvec)

  NB = N1 // R
  g, dinv = pl.pallas_call(
      _mm_scale_body,
      grid=(NB,),
      in_specs=[
          pl.BlockSpec((R, D), lambda i: (i, 0)),
          pl.BlockSpec((D, H), lambda i: (0, 0)),
          pl.BlockSpec((NC, R, 1), lambda i: (0, i, 0)),
      ],
      out_specs=[
          pl.BlockSpec((R, H), lambda i: (i, 0)),
          pl.BlockSpec((R, 1), lambda i: (i, 0)),
      ],
      out_shape=[
          jax.ShapeDtypeStruct((N1, H), jnp.float32),
          jax.ShapeDtypeStruct((N1, 1), jnp.float32),
      ],
  )(xp, W1, deg_part.reshape(NC, N1, 1))

  acc_part, t_part = _make_prop_kernel(N1, N1a, H, NC, NS, CPW)(
      src3, dst3, g, dinv.reshape(N1), zrows, zvec)

  NBa = N1a // Ra
  v, out = pl.pallas_call(
      functools.partial(_combine_body, N, NBa),
      grid=(NBa,),
      in_specs=[
          pl.BlockSpec((NC, Ra, H), lambda i: (0, i, 0)),
          pl.BlockSpec((Ra, H), lambda i: (i, 0)),
          pl.BlockSpec((Ra, 1), lambda i: (i, 0)),
          pl.BlockSpec((NC, Ra, 1), lambda i: (0, i, 0)),
          pl.BlockSpec((1, H), lambda i: (0, 0)),
          pl.BlockSpec((H, H), lambda i: (0, 0)),
          pl.BlockSpec((1, H), lambda i: (0, 0)),
      ],
      out_specs=[
          pl.BlockSpec((1, H), lambda i: (0, 0)),
          pl.BlockSpec((1, H), lambda i: (0, 0)),
      ],
      out_shape=[
          jax.ShapeDtypeStruct((1, H), jnp.float32),
          jax.ShapeDtypeStruct((1, H), jnp.float32),
      ],
  )(acc_part, g, dinv, t_part.reshape(NC, N1, 1),
    b1.reshape(1, H), W2, b2.reshape(1, H))

  return out.reshape(H)


# speedup vs baseline: 3.0298x; 3.0298x over previous
import functools

import jax
import jax.numpy as jnp
from jax import lax
from jax.experimental import pallas as pl
from jax.experimental.pallas import tpu as pltpu
from jax.experimental.pallas import tpu_sc as plsc

CHUNK = 64
DCHUNK = 128


def _make_deg_kernel(N1, NC, NS, CPW):
  NW = NC * NS
  stripe = N1 // NS
  mesh = plsc.VectorSubcoreMesh(core_axis_name="c", subcore_axis_name="s")

  @functools.partial(
      pl.kernel,
      out_type=jax.ShapeDtypeStruct((NC, N1), jnp.float32),
      mesh=mesh,
      scratch_types=[
          pltpu.VMEM((CPW, DCHUNK), jnp.int32),
          pltpu.VMEM((DCHUNK,), jnp.float32),
          pltpu.VMEM_SHARED((N1,), jnp.float32),
      ],
  )
  def deg_kernel(dst_hbm, zvec_hbm, deg_out, idx_v, ones_v, deg_sh):
    cid = lax.axis_index("c")
    sid = lax.axis_index("s")
    wid = sid * NC + cid
    for i in range(DCHUNK // 16):
      ones_v[pl.ds(i * 16, 16)] = jnp.ones((16,), jnp.float32)
    pltpu.sync_copy(zvec_hbm.at[pl.ds(sid * stripe, stripe)],
                    deg_sh.at[pl.ds(sid * stripe, stripe)])
    pltpu.sync_copy(dst_hbm.at[wid], idx_v)
    plsc.subcore_barrier()

    def body(j, carry):
      pltpu.sync_copy(ones_v, deg_sh.at[idx_v.at[j]], add=True)
      return carry

    lax.fori_loop(0, CPW, body, 0)
    plsc.subcore_barrier()
    pltpu.sync_copy(deg_sh.at[pl.ds(sid * stripe, stripe)],
                    deg_out.at[cid, pl.ds(sid * stripe, stripe)])

  return deg_kernel


def _make_prop_kernel(N1, N1a, H, NC, NS, CPW):
  NW = NC * NS
  stripe = N1 // NS
  stripe_a = N1a // NS
  assert CPW % 16 == 0
  HCPW = CPW // 4
  mesh = plsc.VectorSubcoreMesh(core_axis_name="c", subcore_axis_name="s")

  @functools.partial(
      pl.kernel,
      out_type=(
          jax.ShapeDtypeStruct((NC, N1a, H), jnp.float32),
          jax.ShapeDtypeStruct((NC, N1), jnp.float32),
      ),
      mesh=mesh,
      scratch_types=[
          pltpu.VMEM((HCPW, CHUNK), jnp.int32),
          pltpu.VMEM((HCPW, CHUNK), jnp.int32),
          pltpu.VMEM((CHUNK, H), jnp.float32),
          pltpu.VMEM((CHUNK, H), jnp.float32),
          pltpu.VMEM((CHUNK, H), jnp.float32),
          pltpu.VMEM((CHUNK, H), jnp.float32),
          pltpu.VMEM((HCPW, CHUNK), jnp.float32),
          pltpu.VMEM_SHARED((N1a, H), jnp.float32),
          pltpu.VMEM_SHARED((N1,), jnp.float32),
          [pltpu.SemaphoreType.DMA] * 4,
          [pltpu.SemaphoreType.DMA] * 4,
          [pltpu.SemaphoreType.DMA] * 2,
          pltpu.SemaphoreType.DMA,
      ],
  )
  def prop_kernel(src_hbm, dst_hbm, g_hbm, dinv_hbm, zrows_hbm, zvec_hbm,
                  acc_out, t_out, src_v, dst_v, rows0, rows1, rows2, rows3,
                  dvals_v, acc_sh, t_sh, gsems, ssems, dsems, tsem):
    cid = lax.axis_index("c")
    sid = lax.axis_index("s")
    wid = sid * NC + cid
    pltpu.sync_copy(zrows_hbm.at[pl.ds(sid * stripe_a, stripe_a)],
                    acc_sh.at[pl.ds(sid * stripe_a, stripe_a)])
    pltpu.sync_copy(zvec_hbm.at[pl.ds(sid * stripe, stripe)],
                    t_sh.at[pl.ds(sid * stripe, stripe)])
    plsc.subcore_barrier()

    rows = (rows0, rows1, rows2, rows3)

    for h in range(4):
      pltpu.sync_copy(src_hbm.at[wid, pl.ds(h * HCPW, HCPW)], src_v)
      pltpu.sync_copy(dst_hbm.at[wid, pl.ds(h * HCPW, HCPW)], dst_v)
      for p in range(2):
        pltpu.async_copy(dinv_hbm.at[dst_v.at[p]], dvals_v.at[p], dsems[p])
        pltpu.async_copy(g_hbm.at[src_v.at[p]], rows[p], gsems[p])

      def body(i, carry):
        for p in range(4):
          j = 4 * i + p
          q = (p + 2) % 4
          pltpu.make_async_copy(dinv_hbm.at[dst_v.at[j]], dvals_v.at[j],
                                dsems[p % 2]).wait()
          pltpu.async_copy(dvals_v.at[j], t_sh.at[src_v.at[j]], tsem,
                           add=True)

          @pl.when(j + 2 < HCPW)
          def _():
            pltpu.async_copy(dinv_hbm.at[dst_v.at[j + 2]], dvals_v.at[j + 2],
                             dsems[p % 2])

          pltpu.make_async_copy(g_hbm.at[src_v.at[j]], rows[p],
                                gsems[p]).wait()
          pltpu.async_copy(rows[p], acc_sh.at[dst_v.at[j]], ssems[p],
                           add=True)

          @pl.when(j + 2 < HCPW)
          def _():
            @pl.when(j >= 2)
            def _():
              pltpu.make_async_copy(rows[q], acc_sh.at[dst_v.at[j]],
                                    ssems[q]).wait()

            pltpu.async_copy(g_hbm.at[src_v.at[j + 2]], rows[q], gsems[q])

        return carry

      lax.fori_loop(0, HCPW // 4, body, 0)

      for p in range(4):
        pltpu.make_async_copy(rows[p], acc_sh.at[dst_v.at[0]],
                              ssems[p]).wait()

      def drain(jj, carry):
        pltpu.make_async_copy(zvec_hbm.at[pl.ds(0, CHUNK)], dvals_v.at[0],
                              tsem).wait()
        return carry

      lax.fori_loop(0, HCPW, drain, 0)

    plsc.subcore_barrier()
    pltpu.sync_copy(acc_sh.at[pl.ds(sid * stripe_a, stripe_a)],
                    acc_out.at[cid, pl.ds(sid * stripe_a, stripe_a)])
    pltpu.sync_copy(t_sh.at[pl.ds(sid * stripe, stripe)],
                    t_out.at[cid, pl.ds(sid * stripe, stripe)])

  return prop_kernel


def _mm_scale_body(x_ref, w_ref, degp_ref, g_ref, dinv_ref):
  deg = degp_ref[0] + degp_ref[1] + 1.0
  dinv = lax.rsqrt(deg)
  h = jnp.dot(x_ref[...], w_ref[...], preferred_element_type=jnp.float32,
              precision=lax.Precision.HIGHEST)
  g_ref[...] = h * dinv
  dinv_ref[...] = dinv


def _combine_body(N, NB, accp_ref, g_ref, dinv_ref, tp_ref, b1_ref, w2_ref,
                  b2_ref, v_ref, out_ref):
  i = pl.program_id(0)
  R = g_ref.shape[0]
  dinv = dinv_ref[...]
  accsum = accp_ref[0] + accp_ref[1]
  out1 = dinv * (accsum + g_ref[...]) + b1_ref[...]
  h1r = jnp.maximum(out1, 0.0)
  t = tp_ref[0] + tp_ref[1]
  c = dinv * t + dinv * dinv
  rowid = i * R + lax.broadcasted_iota(jnp.int32, (R, 1), 0)
  c = jnp.where(rowid < N, c, 0.0)
  contrib = jnp.sum(c * h1r, axis=0, keepdims=True)

  @pl.when(i == 0)
  def _():
    v_ref[...] = jnp.zeros_like(v_ref)

  v_ref[...] += contrib

  @pl.when(i == NB - 1)
  def _():
    out_ref[...] = jnp.dot(v_ref[...] * (1.0 / N), w2_ref[...],
                           preferred_element_type=jnp.float32,
                           precision=lax.Precision.HIGHEST) + b2_ref[...]


def kernel(x, edge_index, W1, b1, W2, b2):
  N, D = x.shape
  H = W1.shape[1]
  E = edge_index.shape[1]

  info = plsc.get_sparse_core_info()
  NC, NS = info.num_cores, info.num_subcores
  NW = NC * NS
  CPW = -(-E // (NW * CHUNK))
  CPW = -(-CPW // 16) * 16
  E_pad = NW * CHUNK * CPW
  CPWD = E_pad // (NW * DCHUNK)
  assert CPWD * NW * DCHUNK == E_pad

  R = 1280
  N1 = -(-(N + 1) // R) * R
  assert N1 % (NS * CHUNK) == 0
  Ra = 1264
  N1a = -(-(N + 1) // Ra) * Ra
  assert N1a % (NS * 8) == 0 and N1a <= N1

  src = edge_index[0].astype(jnp.int32)
  dst = edge_index[1].astype(jnp.int32)
  npad = E_pad - E
  pad_dst = N + jnp.arange(npad, dtype=jnp.int32) % (N1a - N)
  pad_src = N + jnp.arange(npad, dtype=jnp.int32) % (N1 - N)
  src3 = jnp.concatenate([src, pad_src]).reshape(NW, CPW, CHUNK)
  dst3 = jnp.concatenate([dst, pad_dst]).reshape(NW, CPW, CHUNK)
  zvec = jnp.zeros((N1,), jnp.float32)
  zrows = jnp.zeros((N1a, H), jnp.float32)
  xp = jnp.concatenate([x, jnp.zeros((N1 - N, D), x.dtype)], axis=0)

  dst3d = jnp.concatenate([dst, pad_dst]).reshape(NW, CPWD, DCHUNK)
  deg_part = _make_deg_kernel(N1, NC, NS, CPWD)(dst3d, zvec)

  NB = N1 // R
  g, dinv = pl.pallas_call(
      _mm_scale_body,
      grid=(NB,),
      in_specs=[
          pl.BlockSpec((R, D), lambda i: (i, 0)),
          pl.BlockSpec((D, H), lambda i: (0, 0)),
          pl.BlockSpec((NC, R, 1), lambda i: (0, i, 0)),
      ],
      out_specs=[
          pl.BlockSpec((R, H), lambda i: (i, 0)),
          pl.BlockSpec((R, 1), lambda i: (i, 0)),
      ],
      out_shape=[
          jax.ShapeDtypeStruct((N1, H), jnp.float32),
          jax.ShapeDtypeStruct((N1, 1), jnp.float32),
      ],
  )(xp, W1, deg_part.reshape(NC, N1, 1))

  acc_part, t_part = _make_prop_kernel(N1, N1a, H, NC, NS, CPW)(
      src3, dst3, g, dinv.reshape(N1), zrows, zvec)

  NBa = N1a // Ra
  v, out = pl.pallas_call(
      functools.partial(_combine_body, N, NBa),
      grid=(NBa,),
      in_specs=[
          pl.BlockSpec((NC, Ra, H), lambda i: (0, i, 0)),
          pl.BlockSpec((Ra, H), lambda i: (i, 0)),
          pl.BlockSpec((Ra, 1), lambda i: (i, 0)),
          pl.BlockSpec((NC, Ra, 1), lambda i: (0, i, 0)),
          pl.BlockSpec((1, H), lambda i: (0, 0)),
          pl.BlockSpec((H, H), lambda i: (0, 0)),
          pl.BlockSpec((1, H), lambda i: (0, 0)),
      ],
      out_specs=[
          pl.BlockSpec((1, H), lambda i: (0, 0)),
          pl.BlockSpec((1, H), lambda i: (0, 0)),
      ],
      out_shape=[
          jax.ShapeDtypeStruct((1, H), jnp.float32),
          jax.ShapeDtypeStruct((1, H), jnp.float32),
      ],
  )(acc_part, g, dinv, t_part.reshape(NC, N1, 1),
    b1.reshape(1, H), W2, b2.reshape(1, H))

  return out.reshape(H)
